# Initial kernel scaffold; baseline (speedup 1.0000x reference)
import jax
import jax.numpy as jnp
from jax.experimental import pallas as pl


def kernel(mentions, W0, b0, W1, b1, W2, b2):
    raise NotImplementedError("write your pallas kernel here")


# speedup vs baseline: 2.1764x; 2.1764x over previous
import jax
import jax.numpy as jnp
from jax.experimental import pallas as pl
from jax.experimental.pallas import tpu as pltpu

N = 4096
D = 1024
K = 50
LW = 64
BLK = 256
MVC = 128


def _layer_body(x_ref, w_ref, b_ref, o_ref):
    h = jax.lax.dot_general(
        x_ref[...], w_ref[...], dimension_numbers=(((1,), (1,)), ((), ())),
        preferred_element_type=jnp.float32)
    h = h + b_ref[...]
    o_ref[...] = jnp.where(h >= 0, h, 0.01 * h)


def _matvec_body(x_ref, w_ref, b2_ref, o_ref):
    o_ref[...] = jax.lax.dot_general(
        x_ref[...], w_ref[...], dimension_numbers=(((1,), (1,)), ((), ())),
        preferred_element_type=jnp.float32) + b2_ref[0, 0]


def _scan_body(s_smem, outv_ref, outi_ref):
    lane = jax.lax.broadcasted_iota(jnp.int32, (1, LW), 1)
    lane0 = lane == 0
    neginf = jnp.float32(-jnp.inf)

    vals0 = jnp.full((1, LW), neginf, dtype=jnp.float32)
    idxs0 = jnp.zeros((1, LW), dtype=jnp.int32)
    outv_ref[0:1, :] = vals0
    outi_ref[0:1, :] = lane

    def step(i, carry):
        vals, idxs = carry
        v = s_smem[i - 1]
        keep = vals >= v
        rv = pltpu.roll(vals, 1, 1)
        ri = pltpu.roll(idxs, 1, 1)
        rk = pltpu.roll(keep.astype(jnp.int32), 1, 1) > 0
        ins = jnp.logical_or(rk, lane0)
        vals = jnp.where(keep, vals, jnp.where(ins, v, rv))
        idxs = jnp.where(keep, idxs, jnp.where(ins, i - 1, ri))
        fin_idx = jnp.where(vals == neginf, lane, idxs)
        outv_ref[pl.ds(i, 1), :] = s_smem[i] + vals
        outi_ref[pl.ds(i, 1), :] = fin_idx
        return vals, idxs

    jax.lax.fori_loop(1, N, step, (vals0, idxs0), unroll=False)


def _layer(x, W, b):
    return pl.pallas_call(
        _layer_body,
        grid=(N // BLK,),
        in_specs=[
            pl.BlockSpec((BLK, D), lambda m: (m, 0)),
            pl.BlockSpec((D, D), lambda m: (0, 0)),
            pl.BlockSpec((1, D), lambda m: (0, 0)),
        ],
        out_specs=pl.BlockSpec((BLK, D), lambda m: (m, 0)),
        out_shape=jax.ShapeDtypeStruct((N, D), jnp.float32),
    )(x, W, b.reshape(1, D))


@jax.jit
def kernel(mentions, W0, b0, W1, b1, W2, b2):
    h = _layer(mentions, W0, b0)
    h = _layer(h, W1, b1)

    w2pad = jnp.zeros((MVC, D), jnp.float32).at[0].set(W2[0])
    sv = pl.pallas_call(
        _matvec_body,
        grid=(N // BLK,),
        in_specs=[
            pl.BlockSpec((BLK, D), lambda m: (m, 0)),
            pl.BlockSpec((MVC, D), lambda m: (0, 0)),
            pl.BlockSpec(memory_space=pltpu.SMEM),
        ],
        out_specs=pl.BlockSpec((BLK, MVC), lambda m: (m, 0)),
        out_shape=jax.ShapeDtypeStruct((N, MVC), jnp.float32),
    )(h, w2pad, b2.reshape(1, 1))

    s_flat = sv[:, 0]
    outv, outi = pl.pallas_call(
        _scan_body,
        in_specs=[
            pl.BlockSpec(memory_space=pltpu.SMEM),
        ],
        out_specs=[
            pl.BlockSpec((N, LW), lambda: (0, 0)),
            pl.BlockSpec((N, LW), lambda: (0, 0)),
        ],
        out_shape=[
            jax.ShapeDtypeStruct((N, LW), jnp.float32),
            jax.ShapeDtypeStruct((N, LW), jnp.int32),
        ],
    )(s_flat)

    return outv[:, :K], outi[:, :K]


# speedup vs baseline: 2.8556x; 1.3121x over previous
import jax
import jax.numpy as jnp
from jax.experimental import pallas as pl
from jax.experimental.pallas import tpu as pltpu

N = 4096
D = 1024
K = 50
LW = 64
BLK = 256
MVC = 128


def _layer_body(x_ref, w_ref, b_ref, o_ref):
    h = jax.lax.dot_general(
        x_ref[...], w_ref[...], dimension_numbers=(((1,), (1,)), ((), ())),
        preferred_element_type=jnp.float32)
    h = h + b_ref[...]
    o_ref[...] = jnp.where(h >= 0, h, 0.01 * h)


def _matvec_body(x_ref, w_ref, b2_ref, o_ref):
    o_ref[...] = jax.lax.dot_general(
        x_ref[...], w_ref[...], dimension_numbers=(((1,), (1,)), ((), ())),
        preferred_element_type=jnp.float32) + b2_ref[0, 0]


SCAN_B = 8


def _scan_body(s_smem, outv_ref, outi_ref, vscr, iscr):
    lane = jax.lax.broadcasted_iota(jnp.int32, (1, LW), 1)
    lane0 = lane == 0
    neginf = jnp.float32(-jnp.inf)
    posinf = jnp.float32(jnp.inf)
    pid = pl.program_id(0)
    base = pid * SCAN_B

    @pl.when(pid == 0)
    def _():
        vscr[...] = jnp.full((1, LW), neginf, dtype=jnp.float32)
        iscr[...] = jnp.zeros((1, LW), dtype=jnp.int32)

    vals = vscr[...]
    idxs = iscr[...]
    rows_v = []
    rows_i = []
    for m in range(SCAN_B):
        i = base + m
        iprev = jnp.maximum(i - 1, 0) if m == 0 else i - 1
        v = s_smem[iprev]
        keep = vals >= v
        rv = jnp.where(lane0, posinf, pltpu.roll(vals, 1, 1))
        ri = pltpu.roll(idxs, 1, 1)
        ins = rv >= v
        nv = jnp.where(keep, vals, jnp.minimum(v, rv))
        ni = jnp.where(keep, idxs, jnp.where(ins, i - 1, ri))
        if m == 0:
            nv = jnp.where(pid > 0, nv, vals)
            ni = jnp.where(pid > 0, ni, idxs)
        vals, idxs = nv, ni
        rows_v.append(s_smem[i] + vals)
        rows_i.append(jnp.where(vals == neginf, lane, idxs))
    vscr[...] = vals
    iscr[...] = idxs
    outv_ref[...] = jnp.concatenate(rows_v, axis=0)
    outi_ref[...] = jnp.concatenate(rows_i, axis=0)


def _layer(x, W, b):
    return pl.pallas_call(
        _layer_body,
        grid=(N // BLK,),
        in_specs=[
            pl.BlockSpec((BLK, D), lambda m: (m, 0)),
            pl.BlockSpec((D, D), lambda m: (0, 0)),
            pl.BlockSpec((1, D), lambda m: (0, 0)),
        ],
        out_specs=pl.BlockSpec((BLK, D), lambda m: (m, 0)),
        out_shape=jax.ShapeDtypeStruct((N, D), jnp.float32),
    )(x, W, b.reshape(1, D))


@jax.jit
def kernel(mentions, W0, b0, W1, b1, W2, b2):
    h = _layer(mentions, W0, b0)
    h = _layer(h, W1, b1)

    w2pad = jnp.zeros((MVC, D), jnp.float32).at[0].set(W2[0])
    sv = pl.pallas_call(
        _matvec_body,
        grid=(N // BLK,),
        in_specs=[
            pl.BlockSpec((BLK, D), lambda m: (m, 0)),
            pl.BlockSpec((MVC, D), lambda m: (0, 0)),
            pl.BlockSpec(memory_space=pltpu.SMEM),
        ],
        out_specs=pl.BlockSpec((BLK, MVC), lambda m: (m, 0)),
        out_shape=jax.ShapeDtypeStruct((N, MVC), jnp.float32),
    )(h, w2pad, b2.reshape(1, 1))

    s_flat = sv[:, 0]
    outv, outi = pl.pallas_call(
        _scan_body,
        grid=(N // SCAN_B,),
        in_specs=[
            pl.BlockSpec(memory_space=pltpu.SMEM),
        ],
        out_specs=[
            pl.BlockSpec((SCAN_B, LW), lambda m: (m, 0)),
            pl.BlockSpec((SCAN_B, LW), lambda m: (m, 0)),
        ],
        out_shape=[
            jax.ShapeDtypeStruct((N, LW), jnp.float32),
            jax.ShapeDtypeStruct((N, LW), jnp.int32),
        ],
        scratch_shapes=[
            pltpu.VMEM((1, LW), jnp.float32),
            pltpu.VMEM((1, LW), jnp.int32),
        ],
    )(s_flat)

    return outv[:, :K], outi[:, :K]


# speedup vs baseline: 22.3219x; 7.8170x over previous
import jax
import jax.numpy as jnp
from jax import lax
from jax.experimental import pallas as pl
from jax.experimental.pallas import tpu as pltpu
from jax.experimental.pallas import tpu_sc as plsc

N = 4096
D = 1024
K = 50
LW = 64
BLK = 256
MVC = 128


def _layer_body(x_ref, w_ref, b_ref, o_ref):
    h = jax.lax.dot_general(
        x_ref[...], w_ref[...], dimension_numbers=(((1,), (1,)), ((), ())),
        preferred_element_type=jnp.float32)
    h = h + b_ref[...]
    o_ref[...] = jnp.where(h >= 0, h, 0.01 * h)


def _matvec_body(x_ref, w_ref, b2_ref, o_ref):
    o_ref[...] = jax.lax.dot_general(
        x_ref[...], w_ref[...], dimension_numbers=(((1,), (1,)), ((), ())),
        preferred_element_type=jnp.float32) + b2_ref[0, 0]


NW = 32
CH = N // NW
SCL = 16


def _sel_body(s_hbm, outv_hbm, outi_hbm, s_v, lvp, lip, gsv, gsi, bv, bi):
    wid = lax.axis_index("c") * 16 + lax.axis_index("s")
    base = wid * CH
    neginf = jnp.float32(-jnp.inf)
    posinf = jnp.float32(jnp.inf)

    pltpu.sync_copy(s_hbm, s_v.at[pl.ds(0, N)])

    def _lmin():
        return lvp[pl.ds(64, SCL)][0]

    lane16 = lax.broadcasted_iota(jnp.int32, (SCL,), 0)
    lvp[pl.ds(0, SCL)] = jnp.where(lane16 == 0, posinf, neginf)
    lip[pl.ds(0, SCL)] = jnp.zeros((SCL,), jnp.int32)
    for q in range(1, 5):
        lvp[pl.ds(q * SCL, SCL)] = jnp.full((SCL,), neginf, jnp.float32)
        lip[pl.ds(q * SCL, SCL)] = jnp.zeros((SCL,), jnp.int32)

    def _insert(v, j):
        cur = [lvp[pl.ds(1 + SCL * q, SCL)] for q in range(4)]
        prv = [lvp[pl.ds(SCL * q, SCL)] for q in range(4)]
        curi = [lip[pl.ds(1 + SCL * q, SCL)] for q in range(4)]
        prvi = [lip[pl.ds(SCL * q, SCL)] for q in range(4)]
        for q in range(4):
            keep = jnp.logical_or(
                cur[q] > v, jnp.logical_and(cur[q] == v, curi[q] < j))
            ins = jnp.logical_or(
                prv[q] > v, jnp.logical_and(prv[q] == v, prvi[q] < j))
            nv = jnp.where(keep, cur[q], jnp.minimum(prv[q], v))
            ni = jnp.where(keep, curi[q], jnp.where(ins, j, prvi[q]))
            lvp[pl.ds(1 + SCL * q, SCL)] = nv
            lip[pl.ds(1 + SCL * q, SCL)] = ni

    def group_body(g, carry):
        vec = s_v[pl.ds(g * SCL, SCL)]
        sk, sv_idx = plsc.sort_key_val(vec, lane16 + g * SCL, descending=True)
        gsv[pl.ds(0, SCL)] = sk
        gsi[pl.ds(0, SCL)] = sv_idx

        def cond(e):
            return jnp.logical_and(e < SCL, gsv[pl.ds(e, SCL)][0] >= _lmin())

        def body(e):
            _insert(gsv[pl.ds(e, SCL)][0], gsi[pl.ds(e, SCL)][0])
            return e + 1

        lax.while_loop(cond, body, 0)
        return carry

    lax.fori_loop(0, base // SCL, group_body, 0)

    def row_body(r, carry):
        i = base + r
        si = s_v[pl.ds(i, SCL)][0]
        for q in range(4):
            vq = lvp[pl.ds(1 + SCL * q, SCL)]
            iq = lip[pl.ds(1 + SCL * q, SCL)]
            fin = jnp.where(vq == neginf, lane16 + SCL * q, iq)
            bv[pl.ds(r * LW + SCL * q, SCL)] = si + vq
            bi[pl.ds(r * LW + SCL * q, SCL)] = fin

        @pl.when(si > _lmin())
        def _():
            _insert(si, i)

        return carry

    lax.fori_loop(0, CH, row_body, 0)

    pltpu.sync_copy(bv, outv_hbm.at[pl.ds(base * LW, CH * LW)])
    pltpu.sync_copy(bi, outi_hbm.at[pl.ds(base * LW, CH * LW)])


SCAN_B = 8


def _scan_body(s_smem, outv_ref, outi_ref, vscr, iscr):
    lane = jax.lax.broadcasted_iota(jnp.int32, (1, LW), 1)
    lane0 = lane == 0
    neginf = jnp.float32(-jnp.inf)
    posinf = jnp.float32(jnp.inf)
    pid = pl.program_id(0)
    base = pid * SCAN_B

    @pl.when(pid == 0)
    def _():
        vscr[...] = jnp.full((1, LW), neginf, dtype=jnp.float32)
        iscr[...] = jnp.zeros((1, LW), dtype=jnp.int32)

    vals = vscr[...]
    idxs = iscr[...]
    rows_v = []
    rows_i = []
    for m in range(SCAN_B):
        i = base + m
        iprev = jnp.maximum(i - 1, 0) if m == 0 else i - 1
        v = s_smem[iprev]
        keep = vals >= v
        rv = jnp.where(lane0, posinf, pltpu.roll(vals, 1, 1))
        ri = pltpu.roll(idxs, 1, 1)
        ins = rv >= v
        nv = jnp.where(keep, vals, jnp.minimum(v, rv))
        ni = jnp.where(keep, idxs, jnp.where(ins, i - 1, ri))
        if m == 0:
            nv = jnp.where(pid > 0, nv, vals)
            ni = jnp.where(pid > 0, ni, idxs)
        vals, idxs = nv, ni
        rows_v.append(s_smem[i] + vals)
        rows_i.append(jnp.where(vals == neginf, lane, idxs))
    vscr[...] = vals
    iscr[...] = idxs
    outv_ref[...] = jnp.concatenate(rows_v, axis=0)
    outi_ref[...] = jnp.concatenate(rows_i, axis=0)


def _layer(x, W, b):
    return pl.pallas_call(
        _layer_body,
        grid=(N // BLK,),
        in_specs=[
            pl.BlockSpec((BLK, D), lambda m: (m, 0)),
            pl.BlockSpec((D, D), lambda m: (0, 0)),
            pl.BlockSpec((1, D), lambda m: (0, 0)),
        ],
        out_specs=pl.BlockSpec((BLK, D), lambda m: (m, 0)),
        out_shape=jax.ShapeDtypeStruct((N, D), jnp.float32),
    )(x, W, b.reshape(1, D))


@jax.jit
def kernel(mentions, W0, b0, W1, b1, W2, b2):
    h = _layer(mentions, W0, b0)
    h = _layer(h, W1, b1)

    w2pad = jnp.zeros((MVC, D), jnp.float32).at[0].set(W2[0])
    sv = pl.pallas_call(
        _matvec_body,
        grid=(N // BLK,),
        in_specs=[
            pl.BlockSpec((BLK, D), lambda m: (m, 0)),
            pl.BlockSpec((MVC, D), lambda m: (0, 0)),
            pl.BlockSpec(memory_space=pltpu.SMEM),
        ],
        out_specs=pl.BlockSpec((BLK, MVC), lambda m: (m, 0)),
        out_shape=jax.ShapeDtypeStruct((N, MVC), jnp.float32),
    )(h, w2pad, b2.reshape(1, 1))

    s_flat = sv[:, 0]
    sel = pl.kernel(
        _sel_body,
        out_type=[
            jax.ShapeDtypeStruct((N * LW,), jnp.float32),
            jax.ShapeDtypeStruct((N * LW,), jnp.int32),
        ],
        mesh=plsc.VectorSubcoreMesh(core_axis_name="c", subcore_axis_name="s"),
        compiler_params=pltpu.CompilerParams(needs_layout_passes=False),
        scratch_types=[
            pltpu.VMEM((N + SCL,), jnp.float32),
            pltpu.VMEM((80,), jnp.float32),
            pltpu.VMEM((80,), jnp.int32),
            pltpu.VMEM((2 * SCL,), jnp.float32),
            pltpu.VMEM((2 * SCL,), jnp.int32),
            pltpu.VMEM((CH * LW,), jnp.float32),
            pltpu.VMEM((CH * LW,), jnp.int32),
        ],
    )
    outv, outi = sel(s_flat)
    outv = outv.reshape(N, LW)
    outi = outi.reshape(N, LW)

    return outv[:, :K], outi[:, :K]
